# Initial kernel scaffold; baseline (speedup 1.0000x reference)
import jax
import jax.numpy as jnp
from jax.experimental import pallas as pl


def kernel(update_values, update_features, update_indices, update_feature_indices, filter_indices, update_weights, update_indices_empty, update_weights_empty, values_volume, features_volume, weights_volume, feature_weights_volume):
    raise NotImplementedError("write your pallas kernel here")


# speedup vs baseline: 1.2379x; 1.2379x over previous
import functools
import jax
import jax.numpy as jnp
from jax import lax
from jax.experimental import pallas as pl
from jax.experimental.pallas import tpu as pltpu
from jax.experimental.pallas import tpu_sc as plsc

_MAXW = 500.0


def _sc_coalesce(src, idxs, zeros, n_rows, n_points, half):
    NC, NS = 2, 16
    per_tile = n_points // NS
    n_chunks = per_tile // 128
    wchunk = half // NS
    mesh = plsc.VectorSubcoreMesh(core_axis_name="c", subcore_axis_name="s")

    @functools.partial(
        pl.kernel,
        mesh=mesh,
        out_type=jax.ShapeDtypeStruct((n_rows * 2 * half,), jnp.float32),
        scratch_types=[
            pltpu.VMEM_SHARED((half + 8,), jnp.float32),
            pltpu.VMEM((128,), jnp.int32),
            pltpu.VMEM((128,), jnp.float32),
        ],
    )
    def k(src_hbm, idxs_hbm, zeros_hbm, out_hbm, acc, idx_v, src_v):
        cid = lax.axis_index("c")
        sid = lax.axis_index("s")
        base = sid * per_tile
        for p in range(n_rows):
            row = cid * n_rows + p

            @pl.when(sid == 0)
            def _():
                pltpu.sync_copy(zeros_hbm, acc)

            plsc.subcore_barrier()

            def body(j, carry):
                off = base + j * 128
                pltpu.sync_copy(
                    idxs_hbm.at[pl.ds(row * n_points + off, 128)], idx_v)
                pltpu.sync_copy(
                    src_hbm.at[pl.ds(p * n_points + off, 128)], src_v)
                pltpu.sync_copy(src_v, acc.at[idx_v], add=True)
                return carry

            lax.fori_loop(0, n_chunks, body, 0)
            plsc.subcore_barrier()
            pltpu.sync_copy(
                acc.at[pl.ds(sid * wchunk, wchunk)],
                out_hbm.at[pl.ds(p * 2 * half + cid * half + sid * wchunk,
                                 wchunk)],
            )
            plsc.subcore_barrier()

    out = k(src.reshape(-1), idxs.reshape(-1), zeros)
    return out.reshape(n_rows, 2 * half)


def _dense_body(v_ref, w_ref, fw_ref, ft_ref, c_ref,
                nv_ref, nw_ref, nfw_ref, nft_ref):
    v = v_ref[...]
    w = w_ref[...]
    fw = fw_ref[...]
    ft = ft_ref[...]
    cc = c_ref[...]
    occ = cc[0]
    upd = cc[1]
    wc = cc[2]
    occe = cc[3]
    wec = cc[4]
    occf = cc[5]
    wfc = cc[6]
    fs = cc[7:]

    denom = w + wc
    vu = (w * v + upd) / denom
    wu = jnp.clip(w + wc, 0.0, _MAXW)
    denome = w + wec
    vue = (w * v + 0.1 * wec) / denome
    wue = jnp.clip(w + wec, 0.0, _MAXW)
    nv_ref[...] = jnp.where(occ > 0, vu, jnp.where(occe > 0, vue, v))
    nw_ref[...] = jnp.where(occ > 0, wu, jnp.where(occe > 0, wue, w))

    denomf = w + wfc
    fu = (w[None] * ft + fs) / denomf[None]
    nft_ref[...] = jnp.where(occf[None] > 0, fu, ft)
    nfw_ref[...] = jnp.where(occf > 0, jnp.clip(w + wfc, 0.0, _MAXW), fw)


def _dense_update(vflat, wflat, fwflat, feat_t, coal, rows, cols, f4, n_rows):
    BR = 32
    grid = rows // BR
    v2 = vflat.reshape(rows, cols)
    w2 = wflat.reshape(rows, cols)
    fw2 = fwflat.reshape(rows, cols)
    ft3 = feat_t.reshape(f4, rows, cols)
    c3 = coal.reshape(n_rows, rows, cols)
    out_shapes = (
        jax.ShapeDtypeStruct((rows, cols), jnp.float32),
        jax.ShapeDtypeStruct((rows, cols), jnp.float32),
        jax.ShapeDtypeStruct((rows, cols), jnp.float32),
        jax.ShapeDtypeStruct((f4, rows, cols), jnp.float32),
    )
    s2 = pl.BlockSpec((BR, cols), lambda i: (i, 0))
    s3f = pl.BlockSpec((f4, BR, cols), lambda i: (0, i, 0))
    s3c = pl.BlockSpec((n_rows, BR, cols), lambda i: (0, i, 0))
    return pl.pallas_call(
        _dense_body,
        grid=(grid,),
        in_specs=[s2, s2, s2, s3f, s3c],
        out_specs=(s2, s2, s2, s3f),
        out_shape=out_shapes,
    )(v2, w2, fw2, ft3, c3)


def kernel(update_values, update_features, update_indices,
           update_feature_indices, filter_indices, update_weights,
           update_indices_empty, update_weights_empty, values_volume,
           features_volume, weights_volume, feature_weights_volume):
    xs, ys, zs = values_volume.shape
    f4 = update_features.shape[-1]
    M = xs * ys * zs
    H = M // 2
    n = update_weights.size

    values = update_values.reshape(-1).astype(jnp.float32)
    feat = update_features.reshape(-1, f4).astype(jnp.float32)
    w = update_weights.reshape(-1).astype(jnp.float32)
    we = update_weights_empty.reshape(-1).astype(jnp.float32)
    ia = update_indices.reshape(-1, 3).astype(jnp.int32)
    ifa = update_feature_indices.reshape(-1, 3).astype(jnp.int32)
    iea = update_indices_empty.reshape(-1, 3).astype(jnp.int32)

    def lin(ix):
        return ys * zs * ix[:, 0] + zs * ix[:, 1] + ix[:, 2]

    idx = lin(ia)
    idx_e = lin(iea)
    idx_f = lin(ifa)

    ones = jnp.ones((n,), jnp.float32)
    wf_feat = (w[:, None] * feat).T
    src = jnp.concatenate([
        jnp.stack([ones, w * values, w, ones, we, ones, w]),
        wf_feat,
    ], axis=0)
    n_rows = src.shape[0]

    idx_rows = jnp.stack([idx, idx, idx, idx_e, idx_e, idx_f, idx_f]
                         + [idx_f] * f4)
    halves = []
    for h in range(2):
        local = idx_rows - h * H
        ok = (local >= 0) & (local < H)
        halves.append(jnp.where(ok, local, H))
    idxs = jnp.concatenate(halves, axis=0).astype(jnp.int32)
    zeros = jnp.zeros((H + 8,), jnp.float32)

    coal = _sc_coalesce(src, idxs, zeros, n_rows, n, H)

    fvol_t = features_volume.reshape(M, f4).T.reshape(f4, M)

    rows, cols = 2048, M // 2048
    nv, nw, nfw, nft = _dense_update(
        values_volume.reshape(-1), weights_volume.reshape(-1),
        feature_weights_volume.reshape(-1), fvol_t, coal,
        rows, cols, f4, n_rows)

    new_values = nv.reshape(xs, ys, zs)
    new_weights = nw.reshape(xs, ys, zs)
    new_features = nft.reshape(f4, M).T.reshape(xs, ys, zs, f4)
    new_feature_weights = nfw.reshape(xs, ys, zs)
    return (new_values, new_weights, new_features, new_feature_weights)
